# baseline (speedup 1.0000x reference)
import functools

import jax
import jax.numpy as jnp
from jax import lax
from jax.experimental import pallas as pl
from jax.experimental.pallas import tpu as pltpu
from jax.experimental.pallas import tpu_sc as plsc

N_PFAS, N_GW, N_SW = 50000, 100000, 10000
D = 128

NC, NS = 2, 16
NW = NC * NS
K = 128
R_TOT = 9728
R_USE = 8960
ZB = 32
BF = 56


def _ceil_div(a, b):
    return (a + b - 1) // b


def _seg_sums_kernel(n_dst, e_pad):
    n_pass = _ceil_div(n_dst, 2 * R_USE)
    n_rows = n_pass * 2 * R_USE
    nk = e_pad // (NS * K)

    @functools.partial(
        pl.kernel,
        out_type=(
            jax.ShapeDtypeStruct((n_rows, D), jnp.float32),
            jax.ShapeDtypeStruct((n_rows,), jnp.float32),
        ),
        mesh=plsc.VectorSubcoreMesh(core_axis_name="c", subcore_axis_name="s"),
        scratch_types=[
            pltpu.VMEM((K,), jnp.int32),
            pltpu.VMEM((K,), jnp.int32),
            pltpu.VMEM((K,), jnp.int32),
            pltpu.VMEM((K,), jnp.float32),
            pltpu.VMEM((K, D), jnp.float32),
            pltpu.VMEM((ZB, D), jnp.float32),
            pltpu.VMEM((ZB * 16,), jnp.float32),
            pltpu.VMEM((BF, D), jnp.float32),
            pltpu.VMEM((R_USE // NS,), jnp.float32),
            pltpu.VMEM_SHARED((R_TOT, D), jnp.float32),
            pltpu.VMEM_SHARED((R_TOT,), jnp.float32),
            pltpu.SemaphoreType.DMA,
        ],
    )
    def k(x_hbm, si_hbm, di_hbm, sum_hbm, cnt_hbm,
          src_v, dst_v, loc_v, ones_v, rows_v, zf_v, z1_v, fb_v, cb_v,
          acc_sh, cnt_sh, sem):
        c = lax.axis_index("c")
        s = lax.axis_index("s")
        wid = s * NC + c

        def init_ones(j, _):
            ones_v[pl.ds(j * 16, 16)] = jnp.ones((16,), jnp.float32)
            return _
        lax.fori_loop(0, K // 16, init_ones, None)

        def init_z1(j, _):
            z1_v[pl.ds(j * 16, 16)] = jnp.zeros((16,), jnp.float32)
            return _
        lax.fori_loop(0, ZB, init_z1, None)

        def init_zf(j, _):
            for g in range(D // 16):
                zf_v[j, pl.ds(g * 16, 16)] = jnp.zeros((16,), jnp.float32)
            return _
        lax.fori_loop(0, ZB, init_zf, None)

        for p in range(n_pass):
            base = (2 * p + c) * R_USE

            rows_per_tile = R_TOT // NS
            for zi in range(rows_per_tile // ZB):
                r0 = s * rows_per_tile + zi * ZB
                pltpu.sync_copy(zf_v, acc_sh.at[pl.ds(r0, ZB)])
            for zi in range(_ceil_div(rows_per_tile, ZB * 16)):
                r0 = s * rows_per_tile + zi * ZB * 16
                nz = min(ZB * 16, rows_per_tile - zi * ZB * 16)
                pltpu.sync_copy(z1_v.at[pl.ds(0, nz)],
                                cnt_sh.at[pl.ds(r0, nz)])
            plsc.subcore_barrier()

            @pl.loop(0, nk)
            def chunk(g):
                e0 = (s * nk + g) * K
                pltpu.sync_copy(si_hbm.at[pl.ds(e0, K)], src_v)
                pltpu.sync_copy(di_hbm.at[pl.ds(e0, K)], dst_v)
                for j in range(K // 16):
                    d = dst_v[pl.ds(j * 16, 16)]
                    lo = d - base
                    m = (lo >= 0) & (lo < R_USE)
                    loc_v[pl.ds(j * 16, 16)] = jnp.where(m, lo, R_USE + j)
                pltpu.async_copy(x_hbm.at[src_v], rows_v, sem).wait()
                pltpu.sync_copy(rows_v, acc_sh.at[loc_v], add=True)
                pltpu.sync_copy(ones_v, cnt_sh.at[loc_v], add=True)
            plsc.subcore_barrier()

            d_rows = R_USE // NS
            out0 = (2 * p + c) * R_USE + s * d_rows
            for bi in range(d_rows // BF):
                pltpu.sync_copy(
                    acc_sh.at[pl.ds(s * d_rows + bi * BF, BF)], fb_v)
                pltpu.sync_copy(fb_v, sum_hbm.at[pl.ds(out0 + bi * BF, BF)])
            pltpu.sync_copy(cnt_sh.at[pl.ds(s * d_rows, d_rows)], cb_v)
            pltpu.sync_copy(cb_v, cnt_hbm.at[pl.ds(out0, d_rows)])
            plsc.subcore_barrier()

    return k


def _pad_edges(ei, n_src, e_pad):
    src, dst = ei[0], ei[1]
    pad = e_pad - src.shape[0]
    psrc = lax.iota(jnp.int32, pad) % n_src
    pdst = jnp.full((pad,), -1, jnp.int32)
    return jnp.concatenate([src, psrc]), jnp.concatenate([dst, pdst])


_BLK = 400


def _dense(n, x_dst, terms, Wrs, bls, Wp=None, bp=None, out_dim=1):
    nt = len(terms)
    grid = n // _BLK

    def body(*refs):
        irefs = refs[:-1]
        o_ref = refs[-1]
        pos = 0
        acc = None
        for _ in range(nt):
            s_ref, c_ref, wl_ref = irefs[pos], irefs[pos + 1], irefs[pos + 2]
            pos += 3
            mean = s_ref[...] / jnp.maximum(c_ref[...], 1.0)
            t = jnp.dot(mean, wl_ref[...], preferred_element_type=jnp.float32)
            acc = t if acc is None else acc + t
        wr = None
        for _ in range(len(Wrs)):
            w = irefs[pos][...]
            pos += 1
            wr = w if wr is None else wr + w
        x_ref = irefs[pos]
        pos += 1
        acc = acc + jnp.dot(x_ref[...], wr, preferred_element_type=jnp.float32)
        for _ in range(len(bls)):
            acc = acc + irefs[pos][...]
            pos += 1
        h = jnp.maximum(acc, 0.0)
        if Wp is not None:
            o = jnp.dot(h, irefs[pos][...], preferred_element_type=jnp.float32)
            o = o + irefs[pos + 1][...]
            o_ref[...] = jnp.where(o >= 0.0, o, 0.001 * o)
        else:
            o_ref[...] = h

    args = []
    specs = []
    for (sums, cnt, Wl) in terms:
        args += [sums, cnt, Wl]
        specs += [
            pl.BlockSpec((_BLK, D), lambda i: (i, 0)),
            pl.BlockSpec((_BLK, 1), lambda i: (i, 0)),
            pl.BlockSpec((D, D), lambda i: (0, 0)),
        ]
    for w in Wrs:
        args.append(w)
        specs.append(pl.BlockSpec((D, D), lambda i: (0, 0)))
    args.append(x_dst)
    specs.append(pl.BlockSpec((_BLK, D), lambda i: (i, 0)))
    for b in bls:
        args.append(b.reshape(1, D))
        specs.append(pl.BlockSpec((1, D), lambda i: (0, 0)))
    if Wp is not None:
        args += [Wp, bp.reshape(1, 1)]
        specs += [pl.BlockSpec((D, out_dim), lambda i: (0, 0)),
                  pl.BlockSpec((1, out_dim), lambda i: (0, 0))]

    return pl.pallas_call(
        body,
        grid=(grid,),
        in_specs=specs,
        out_specs=pl.BlockSpec((_BLK, out_dim), lambda i: (i, 0)),
        out_shape=jax.ShapeDtypeStruct((n, out_dim), jnp.float32),
    )(*args)


def kernel(x_pfas_sites, x_gw_wells, x_sw_stations, ei_pfas_to_gw,
           ei_gw_to_pfas, ei_pfas_to_sw, ei_sw_to_pfas, Wl_p2g, Wr_p2g,
           bl_p2g, Wl_g2p, Wr_g2p, bl_g2p, Wl_p2s, Wr_p2s, bl_p2s, Wl_s2p,
           Wr_s2p, bl_s2p, W_gw, b_gw, W_sites, b_sites):
    rels = {
        'p2g': (x_pfas_sites, N_PFAS, N_GW, ei_pfas_to_gw),
        'g2p': (x_gw_wells, N_GW, N_PFAS, ei_gw_to_pfas),
        'p2s': (x_pfas_sites, N_PFAS, N_SW, ei_pfas_to_sw),
        's2p': (x_sw_stations, N_SW, N_PFAS, ei_sw_to_pfas),
    }
    sums, cnts = {}, {}
    for nm, (x_src, n_src, n_dst, ei) in rels.items():
        e = ei.shape[1]
        e_pad = _ceil_div(e, NW * K) * (NW * K)
        si, di = _pad_edges(ei, n_src, e_pad)
        k = _seg_sums_kernel(n_dst, e_pad)
        sums[nm], cnt1 = k(x_src, si, di)
        cnts[nm] = cnt1.reshape(-1, 1)

    out_gw = _dense(N_GW, x_gw_wells, [(sums['p2g'], cnts['p2g'], Wl_p2g)],
                    [Wr_p2g], [bl_p2g], Wp=W_gw, bp=b_gw)
    out_pfas = _dense(N_PFAS, x_pfas_sites,
                      [(sums['g2p'], cnts['g2p'], Wl_g2p),
                       (sums['s2p'], cnts['s2p'], Wl_s2p)],
                      [Wr_g2p, Wr_s2p], [bl_g2p, bl_s2p],
                      Wp=W_sites, bp=b_sites)
    h_sw = _dense(N_SW, x_sw_stations, [(sums['p2s'], cnts['p2s'], Wl_p2s)],
                  [Wr_p2s], [bl_p2s], out_dim=D)
    return (out_gw, out_pfas, h_sw)
